# baseline (speedup 1.0000x reference)
import functools

import jax
import jax.numpy as jnp
import numpy as np
from jax.experimental import pallas as pl
from jax.experimental.pallas import tpu as pltpu

B = 1024
D = 64
M = 100000
K = 128
ALPHA = 0.1
BETA = 1e-8
EPS = 1e-3

MT = 2048
NMT = 49
MP = MT * NMT
RT = 256
RG = 8
NEG = -1e30

_INT_MIN = np.int32(-2**31)


def _score_kernel(q_ref, kt_ref, out_ref):
    j = pl.program_id(1)
    sim = jnp.dot(q_ref[:], kt_ref[:], preferred_element_type=jnp.float32)
    col = j * MT + jax.lax.broadcasted_iota(jnp.int32, (RT, MT), 1)
    out_ref[:] = jnp.where(col < M, sim, NEG)


def _monotone_i32(x):
    b = jax.lax.bitcast_convert_type(x, jnp.int32)
    return jnp.where(b >= 0, b, b ^ jnp.int32(0x7FFFFFFF))


def _select_kernel(sim_ref, lp_ref, c2_ref, val_ref, out_ref, key_ref):
    nchunks = MP // MT

    for j in range(nchunks):
        sl = pl.ds(j * MT, MT)
        a = sim_ref[:, sl] + lp_ref[:, sl]
        key_ref[:, sl] = _monotone_i32(a)

    kcount = jnp.int32(K)

    def bit_step(i, t_u):
        bit = jax.lax.shift_left(jnp.int32(1), jnp.int32(31) - i)
        cand_u = t_u | bit
        cand_s = cand_u ^ _INT_MIN
        cnt = jnp.zeros((RG, 1), jnp.int32)
        for j in range(nchunks):
            k = key_ref[:, pl.ds(j * MT, MT)]
            m = (k >= cand_s).astype(jnp.int32)
            cnt = cnt + jnp.sum(m, axis=1, keepdims=True)
        return jnp.where(cnt >= kcount, cand_u, t_u)

    t_u = jax.lax.fori_loop(0, 32, bit_step, jnp.zeros((RG, 1), jnp.int32))
    t_s = t_u ^ _INT_MIN

    s_gt_w = jnp.zeros((RG, 1), jnp.float32)
    s_gt_wv = jnp.zeros((RG, 1), jnp.float32)
    s_eq_w = jnp.zeros((RG, 1), jnp.float32)
    s_eq_wv = jnp.zeros((RG, 1), jnp.float32)
    cnt_gt = jnp.zeros((RG, 1), jnp.float32)
    cnt_eq = jnp.zeros((RG, 1), jnp.float32)
    for j in range(nchunks):
        sl = pl.ds(j * MT, MT)
        k = key_ref[:, sl]
        sim = sim_ref[:, sl]
        w = jnp.exp(sim - 1.0) * c2_ref[:, sl]
        wv = w * val_ref[:, sl]
        gt = (k > t_s).astype(jnp.float32)
        eq = (k == t_s).astype(jnp.float32)
        s_gt_w += jnp.sum(w * gt, axis=1, keepdims=True)
        s_gt_wv += jnp.sum(wv * gt, axis=1, keepdims=True)
        s_eq_w += jnp.sum(w * eq, axis=1, keepdims=True)
        s_eq_wv += jnp.sum(wv * eq, axis=1, keepdims=True)
        cnt_gt += jnp.sum(gt, axis=1, keepdims=True)
        cnt_eq += jnp.sum(eq, axis=1, keepdims=True)

    frac = (jnp.float32(K) - cnt_gt) / jnp.maximum(cnt_eq, 1.0)
    denom = s_gt_w + frac * s_eq_w
    numer = s_gt_wv + frac * s_eq_wv
    out_ref[:] = jnp.clip(numer / denom, EPS, 1.0 - EPS)


@jax.jit
def kernel(q, memory_key, memory_values, memory_hist):
    kt = jnp.pad(memory_key, ((0, MP - M), (0, 0))).T
    lp = jnp.pad(jnp.log(memory_hist + BETA), (0, MP - M)).reshape(1, MP)
    c2 = jnp.pad(ALPHA * memory_hist + BETA, (0, MP - M)).reshape(1, MP)
    val = jnp.pad(memory_values, (0, MP - M)).reshape(1, MP)

    sim = pl.pallas_call(
        _score_kernel,
        grid=(B // RT, NMT),
        in_specs=[
            pl.BlockSpec((RT, D), lambda i, j: (i, 0)),
            pl.BlockSpec((D, MT), lambda i, j: (0, j)),
        ],
        out_specs=pl.BlockSpec((RT, MT), lambda i, j: (i, j)),
        out_shape=jax.ShapeDtypeStruct((B, MP), jnp.float32),
        compiler_params=pltpu.CompilerParams(
            dimension_semantics=("parallel", "arbitrary"),
        ),
    )(q, kt)

    res = pl.pallas_call(
        _select_kernel,
        grid=(B // RG,),
        in_specs=[
            pl.BlockSpec((RG, MP), lambda i: (i, 0)),
            pl.BlockSpec((1, MP), lambda i: (0, 0)),
            pl.BlockSpec((1, MP), lambda i: (0, 0)),
            pl.BlockSpec((1, MP), lambda i: (0, 0)),
        ],
        out_specs=pl.BlockSpec((RG, 1), lambda i: (i, 0)),
        out_shape=jax.ShapeDtypeStruct((B, 1), jnp.float32),
        scratch_shapes=[pltpu.VMEM((RG, MP), jnp.int32)],
        compiler_params=pltpu.CompilerParams(
            dimension_semantics=("parallel",),
        ),
    )(sim, lp, c2, val)

    return res.reshape(B)
